# baseline (speedup 1.0000x reference)
import jax
import jax.numpy as jnp
from jax.experimental import pallas as pl
from jax.experimental.pallas import tpu as pltpu

SIZES = [16, 16, 32, 64] + [128] * 6 + [64, 32, 16, 16]
OFFS = [sum(SIZES[:i]) for i in range(len(SIZES))]
MAX_CHUNK = max(SIZES)
N_SLOTS = 4
PREFETCH = 2


def _body(x_hbm, t_ref, o_hbm, buf, in_sems, out_sems):
    n = len(SIZES)

    t = t_ref[...]
    norms = jnp.sqrt(jnp.sum(t * t, axis=-1, keepdims=True))
    scale = jnp.where(norms > 1.0, 1.0 / (norms + 1e-7), 1.0)
    enc = t * scale

    def fetch(i):
        slot = i % N_SLOTS
        return pltpu.make_async_copy(
            x_hbm.at[pl.ds(OFFS[i], SIZES[i])],
            buf.at[slot, pl.ds(0, SIZES[i])],
            in_sems.at[slot],
        )

    def flush(i):
        slot = i % N_SLOTS
        return pltpu.make_async_copy(
            buf.at[slot, pl.ds(0, SIZES[i])],
            o_hbm.at[pl.ds(OFFS[i], SIZES[i])],
            out_sems.at[slot],
        )

    for i in range(min(PREFETCH, n)):
        fetch(i).start()

    for i in range(n):
        slot = i % N_SLOTS
        fetch(i).wait()
        nxt = i + PREFETCH
        if nxt < n:
            if nxt - N_SLOTS >= 0:
                flush(nxt - N_SLOTS).wait()
            fetch(nxt).start()
        buf[slot, pl.ds(0, SIZES[i])] = buf[slot, pl.ds(0, SIZES[i])] + enc
        flush(i).start()

    for i in range(max(0, n - N_SLOTS), n):
        flush(i).wait()


def kernel(x, table):
    B, S, one, D = x.shape
    x3 = x.reshape(B, S, D)
    out = pl.pallas_call(
        _body,
        in_specs=[
            pl.BlockSpec(memory_space=pltpu.HBM),
            pl.BlockSpec(memory_space=pltpu.VMEM),
        ],
        out_specs=pl.BlockSpec(memory_space=pltpu.HBM),
        out_shape=jax.ShapeDtypeStruct((B, S, D), x.dtype),
        scratch_shapes=[
            pltpu.VMEM((N_SLOTS, MAX_CHUNK, S, D), jnp.float32),
            pltpu.SemaphoreType.DMA((N_SLOTS,)),
            pltpu.SemaphoreType.DMA((N_SLOTS,)),
        ],
    )(x3, table)
    return out.reshape(B, S, one, D)
